# Initial kernel scaffold; baseline (speedup 1.0000x reference)
import jax
import jax.numpy as jnp
from jax.experimental import pallas as pl


def kernel(x, edge_index, edge_attr, params):
    raise NotImplementedError("write your pallas kernel here")


# speedup vs baseline: 2.9464x; 2.9464x over previous
import functools

import jax
import jax.numpy as jnp
from jax import lax
from jax.experimental import pallas as pl
from jax.experimental.pallas import tpu as pltpu
from jax.experimental.pallas import tpu_sc as plsc

N_NODES = 10000
N_EDGES = 320000
NODE_DIM = 128
EDGE_DIM = 16
HID = 48
HID_O = 40
ALPHA = 0.5

CHUNK = 128
N_CHUNKS = N_EDGES // CHUNK
NC, NS = 2, 16
NW = NC * NS
ROWS_PER_TILE = N_NODES // NS

_F32 = jnp.float32


def _nodeproj_body(x_ref, w_ref, pa_ref, pb_ref):
    p = jnp.dot(x_ref[...], w_ref[...], preferred_element_type=_F32)
    pa_ref[...] = p[:, :HID]
    pb_ref[...] = p[:, HID:]


def _node_proj(x, wcat):
    bn = 2000
    return pl.pallas_call(
        _nodeproj_body,
        grid=(N_NODES // bn,),
        in_specs=[
            pl.BlockSpec((bn, NODE_DIM), lambda i: (i, 0)),
            pl.BlockSpec((NODE_DIM, 2 * HID), lambda i: (0, 0)),
        ],
        out_specs=[
            pl.BlockSpec((bn, HID), lambda i: (i, 0)),
            pl.BlockSpec((bn, HID), lambda i: (i, 0)),
        ],
        out_shape=[jax.ShapeDtypeStruct((N_NODES, HID), _F32)] * 2,
    )(x, wcat)


def _edge_body(g_ref, ea_ref, c_ref, b1_ref, w2_ref, b2_ref, out_ref):
    m = jnp.dot(ea_ref[...], c_ref[...], preferred_element_type=_F32)
    h = jnp.maximum(g_ref[...] + m + b1_ref[...], 0.0)
    out_ref[...] = jnp.dot(h, w2_ref[...], preferred_element_type=_F32) + b2_ref[...]


def _edge_mlp(gsum, ea, cp, b1p, w2p, b2p):
    be = 2000
    return pl.pallas_call(
        _edge_body,
        grid=(N_EDGES // be,),
        in_specs=[
            pl.BlockSpec((be, HID), lambda i: (i, 0)),
            pl.BlockSpec((be, EDGE_DIM), lambda i: (i, 0)),
            pl.BlockSpec((EDGE_DIM, HID), lambda i: (0, 0)),
            pl.BlockSpec((1, HID), lambda i: (0, 0)),
            pl.BlockSpec((HID, EDGE_DIM), lambda i: (0, 0)),
            pl.BlockSpec((1, EDGE_DIM), lambda i: (0, 0)),
        ],
        out_specs=pl.BlockSpec((be, EDGE_DIM), lambda i: (i, 0)),
        out_shape=jax.ShapeDtypeStruct((N_EDGES, EDGE_DIM), _F32),
    )(gsum, ea, cp, b1p, w2p, b2p)


def _nodeupd_body(x_ref, aggp_ref, o1x_ref, o1a_ref, ob1_ref, o2_ref, ob2_ref,
                  out_ref):
    agg = aggp_ref[0] + aggp_ref[1]
    t = (jnp.dot(x_ref[...], o1x_ref[...], preferred_element_type=_F32)
         + jnp.dot(agg, o1a_ref[...], preferred_element_type=_F32)
         + ob1_ref[...])
    h = jnp.maximum(t, 0.0)
    delta = jnp.dot(h, o2_ref[...], preferred_element_type=_F32) + ob2_ref[...]
    out_ref[...] = ALPHA * x_ref[...] + (1.0 - ALPHA) * jnp.maximum(delta, 0.0)


def _node_update(x, aggp, o1x, o1a, ob1, o2, ob2):
    bn = 2000
    return pl.pallas_call(
        _nodeupd_body,
        grid=(N_NODES // bn,),
        in_specs=[
            pl.BlockSpec((bn, NODE_DIM), lambda i: (i, 0)),
            pl.BlockSpec((2, bn, EDGE_DIM), lambda i: (0, i, 0)),
            pl.BlockSpec((NODE_DIM, HID_O), lambda i: (0, 0)),
            pl.BlockSpec((EDGE_DIM, HID_O), lambda i: (0, 0)),
            pl.BlockSpec((1, HID_O), lambda i: (0, 0)),
            pl.BlockSpec((HID_O, NODE_DIM), lambda i: (0, 0)),
            pl.BlockSpec((1, NODE_DIM), lambda i: (0, 0)),
        ],
        out_specs=pl.BlockSpec((bn, NODE_DIM), lambda i: (i, 0)),
        out_shape=jax.ShapeDtypeStruct((N_NODES, NODE_DIM), _F32),
    )(x, aggp, o1x, o1a, ob1, o2, ob2)


_MESH = plsc.VectorSubcoreMesh(core_axis_name="c", subcore_axis_name="s")


def _sc_gather_body(ta, tb, dst2d, src2d, out, idx_a, idx_b, buf_a, buf_b,
                    sem_a, sem_b):
    cid = lax.axis_index("c")
    sid = lax.axis_index("s")
    wid = cid * NS + sid
    nloc = (N_CHUNKS - wid + NW - 1) // NW

    def chunk_body(i, carry):
        c = wid + i * NW
        pltpu.sync_copy(dst2d.at[c], idx_a)
        pltpu.sync_copy(src2d.at[c], idx_b)
        cp_a = pltpu.async_copy(ta.at[idx_a], buf_a, sem_a)
        cp_b = pltpu.async_copy(tb.at[idx_b], buf_b, sem_b)
        cp_a.wait()
        cp_b.wait()

        def add_row(j, carry2):
            for k in range(HID // 16):
                s = pl.ds(k * 16, 16)
                buf_a[j, s] = buf_a[j, s] + buf_b[j, s]
            return carry2

        lax.fori_loop(0, CHUNK, add_row, 0, unroll=False)
        pltpu.sync_copy(buf_a, out.at[pl.ds(c * CHUNK, CHUNK)])
        return carry

    lax.fori_loop(0, nloc, chunk_body, 0, unroll=False)


_sc_gather = pl.kernel(
    _sc_gather_body,
    out_type=jax.ShapeDtypeStruct((N_EDGES, HID), _F32),
    mesh=_MESH,
    compiler_params=pltpu.CompilerParams(use_tc_tiling_on_sc=False),
    scratch_types=[
        pltpu.VMEM((CHUNK,), jnp.int32),
        pltpu.VMEM((CHUNK,), jnp.int32),
        pltpu.VMEM((CHUNK, HID), _F32),
        pltpu.VMEM((CHUNK, HID), _F32),
        pltpu.SemaphoreType.DMA,
        pltpu.SemaphoreType.DMA,
    ],
)


def _sc_scatter_body(e_hbm, dst2d, out, acc, idx, buf, zbuf):
    cid = lax.axis_index("c")
    sid = lax.axis_index("s")
    wid = cid * NS + sid
    nloc = (N_CHUNKS - wid + NW - 1) // NW

    def zrow(i, carry):
        zbuf[i, :] = jnp.zeros((EDGE_DIM,), _F32)
        return carry

    lax.fori_loop(0, ROWS_PER_TILE, zrow, 0, unroll=False)
    pltpu.sync_copy(zbuf, acc.at[pl.ds(sid * ROWS_PER_TILE, ROWS_PER_TILE)])
    plsc.subcore_barrier()

    def chunk_body(i, carry):
        c = wid + i * NW
        pltpu.sync_copy(e_hbm.at[pl.ds(c * CHUNK, CHUNK)], buf)
        pltpu.sync_copy(dst2d.at[c], idx)
        pltpu.sync_copy(buf, acc.at[idx], add=True)
        return carry

    lax.fori_loop(0, nloc, chunk_body, 0, unroll=False)
    plsc.subcore_barrier()
    pltpu.sync_copy(acc.at[pl.ds(sid * ROWS_PER_TILE, ROWS_PER_TILE)],
                    out.at[cid, pl.ds(sid * ROWS_PER_TILE, ROWS_PER_TILE)])


_sc_scatter = pl.kernel(
    _sc_scatter_body,
    out_type=jax.ShapeDtypeStruct((NC, N_NODES, EDGE_DIM), _F32),
    mesh=_MESH,
    compiler_params=pltpu.CompilerParams(use_tc_tiling_on_sc=False),
    scratch_types=[
        pltpu.VMEM_SHARED((N_NODES, EDGE_DIM), _F32),
        pltpu.VMEM((CHUNK,), jnp.int32),
        pltpu.VMEM((CHUNK, EDGE_DIM), _F32),
        pltpu.VMEM((ROWS_PER_TILE, EDGE_DIM), _F32),
    ],
)


def _layer(x, ea, dst2d, src2d, p):
    rel, obj = p["rel"], p["obj"]
    pad = HID - rel["W1"].shape[1]
    a = jnp.pad(rel["W1"][:NODE_DIM], ((0, 0), (0, pad)))
    b = jnp.pad(rel["W1"][NODE_DIM:2 * NODE_DIM], ((0, 0), (0, pad)))
    cp = jnp.pad(rel["W1"][2 * NODE_DIM:], ((0, 0), (0, pad)))
    wcat = jnp.concatenate([a, b], axis=1)
    b1p = jnp.pad(rel["b1"], (0, pad)).reshape(1, HID)
    w2p = jnp.pad(rel["W2"], ((0, pad), (0, 0)))
    b2p = rel["b2"].reshape(1, EDGE_DIM)

    pa, pb = _node_proj(x, wcat)
    gsum = _sc_gather(pa, pb, dst2d, src2d)
    e_t = _edge_mlp(gsum, ea, cp, b1p, w2p, b2p)
    aggp = _sc_scatter(e_t, dst2d)
    x_new = _node_update(
        x, aggp,
        obj["W1"][:NODE_DIM], obj["W1"][NODE_DIM:],
        obj["b1"].reshape(1, HID_O), obj["W2"], obj["b2"].reshape(1, NODE_DIM))
    return x_new, e_t


def kernel(x, edge_index, edge_attr, params):
    src2d = edge_index[0].reshape(N_CHUNKS, CHUNK)
    dst2d = edge_index[1].reshape(N_CHUNKS, CHUNK)
    ea = edge_attr
    edge_attrs = [edge_attr]
    for p in params:
        x, ea = _layer(x, ea, dst2d, src2d, p)
        edge_attrs.append(ea)
    return x, ea, tuple(edge_attrs)


# speedup vs baseline: 3.5736x; 1.2129x over previous
import functools

import jax
import jax.numpy as jnp
from jax import lax
from jax.experimental import pallas as pl
from jax.experimental.pallas import tpu as pltpu
from jax.experimental.pallas import tpu_sc as plsc

N_NODES = 10000
N_EDGES = 320000
NODE_DIM = 128
EDGE_DIM = 16
HID = 48
HID_O = 40
ALPHA = 0.5

CHUNK = 128
N_CHUNKS = N_EDGES // CHUNK
NC, NS = 2, 16
NW = NC * NS
ROWS_PER_TILE = N_NODES // NS

_F32 = jnp.float32


def _nodeproj_body(x_ref, w_ref, pa_ref, pb_ref):
    p = jnp.dot(x_ref[...], w_ref[...], preferred_element_type=_F32)
    pa_ref[...] = p[:, :HID]
    pb_ref[...] = p[:, HID:]


def _node_proj(x, wcat):
    bn = 2000
    return pl.pallas_call(
        _nodeproj_body,
        grid=(N_NODES // bn,),
        in_specs=[
            pl.BlockSpec((bn, NODE_DIM), lambda i: (i, 0)),
            pl.BlockSpec((NODE_DIM, 2 * HID), lambda i: (0, 0)),
        ],
        out_specs=[
            pl.BlockSpec((bn, HID), lambda i: (i, 0)),
            pl.BlockSpec((bn, HID), lambda i: (i, 0)),
        ],
        out_shape=[jax.ShapeDtypeStruct((N_NODES, HID), _F32)] * 2,
    )(x, wcat)


def _edge_body(g_ref, ea_ref, c_ref, b1_ref, w2_ref, b2_ref, out_ref):
    m = jnp.dot(ea_ref[...], c_ref[...], preferred_element_type=_F32)
    h = jnp.maximum(g_ref[...] + m + b1_ref[...], 0.0)
    out_ref[...] = jnp.dot(h, w2_ref[...], preferred_element_type=_F32) + b2_ref[...]


def _edge_mlp(gsum, ea, cp, b1p, w2p, b2p):
    be = 2000
    return pl.pallas_call(
        _edge_body,
        grid=(N_EDGES // be,),
        in_specs=[
            pl.BlockSpec((be, HID), lambda i: (i, 0)),
            pl.BlockSpec((be, EDGE_DIM), lambda i: (i, 0)),
            pl.BlockSpec((EDGE_DIM, HID), lambda i: (0, 0)),
            pl.BlockSpec((1, HID), lambda i: (0, 0)),
            pl.BlockSpec((HID, EDGE_DIM), lambda i: (0, 0)),
            pl.BlockSpec((1, EDGE_DIM), lambda i: (0, 0)),
        ],
        out_specs=pl.BlockSpec((be, EDGE_DIM), lambda i: (i, 0)),
        out_shape=jax.ShapeDtypeStruct((N_EDGES, EDGE_DIM), _F32),
    )(gsum, ea, cp, b1p, w2p, b2p)


def _nodeupd_body(x_ref, aggp_ref, o1x_ref, o1a_ref, ob1_ref, o2_ref, ob2_ref,
                  out_ref):
    agg = aggp_ref[0] + aggp_ref[1]
    t = (jnp.dot(x_ref[...], o1x_ref[...], preferred_element_type=_F32)
         + jnp.dot(agg, o1a_ref[...], preferred_element_type=_F32)
         + ob1_ref[...])
    h = jnp.maximum(t, 0.0)
    delta = jnp.dot(h, o2_ref[...], preferred_element_type=_F32) + ob2_ref[...]
    out_ref[...] = ALPHA * x_ref[...] + (1.0 - ALPHA) * jnp.maximum(delta, 0.0)


def _node_update(x, aggp, o1x, o1a, ob1, o2, ob2):
    bn = 2000
    return pl.pallas_call(
        _nodeupd_body,
        grid=(N_NODES // bn,),
        in_specs=[
            pl.BlockSpec((bn, NODE_DIM), lambda i: (i, 0)),
            pl.BlockSpec((2, bn, EDGE_DIM), lambda i: (0, i, 0)),
            pl.BlockSpec((NODE_DIM, HID_O), lambda i: (0, 0)),
            pl.BlockSpec((EDGE_DIM, HID_O), lambda i: (0, 0)),
            pl.BlockSpec((1, HID_O), lambda i: (0, 0)),
            pl.BlockSpec((HID_O, NODE_DIM), lambda i: (0, 0)),
            pl.BlockSpec((1, NODE_DIM), lambda i: (0, 0)),
        ],
        out_specs=pl.BlockSpec((bn, NODE_DIM), lambda i: (i, 0)),
        out_shape=jax.ShapeDtypeStruct((N_NODES, NODE_DIM), _F32),
    )(x, aggp, o1x, o1a, ob1, o2, ob2)


_MESH = plsc.VectorSubcoreMesh(core_axis_name="c", subcore_axis_name="s")

IDXR = 79
KG = 6
NGRP = 78 // KG


def _worker_range():
    cid = lax.axis_index("c")
    sid = lax.axis_index("s")
    wid = cid * NS + sid
    start = (wid * N_CHUNKS) // NW
    nloc = ((wid + 1) * N_CHUNKS) // NW - start
    return cid, sid, start, nloc


def _sc_gather_body(ta, tb, dst2d, src2d, out, idx_a, idx_b,
                    buf_a, buf_b, sem_g, sem_w):
    cid, sid, start, nloc = _worker_range()

    pltpu.sync_copy(dst2d.at[pl.ds(start, IDXR)], idx_a)
    pltpu.sync_copy(src2d.at[pl.ds(start, IDXR)], idx_b)

    def do_adds(b):
        def add_row(j, carry2):
            for k in range(HID // 16):
                s = pl.ds(k * 16, 16)
                buf_a[b, j, s] = buf_a[b, j, s] + buf_b[b, j, s]
            return carry2

        lax.fori_loop(0, CHUNK, add_row, 0, unroll=False)

    def group(g, carry):
        r0 = g * KG
        cps = []
        for b in range(KG):
            cps.append(pltpu.async_copy(ta.at[idx_a.at[r0 + b]],
                                        buf_a.at[b], sem_g))
            cps.append(pltpu.async_copy(tb.at[idx_b.at[r0 + b]],
                                        buf_b.at[b], sem_g))
        for cp in cps:
            cp.wait()
        wps = []
        for b in range(KG):
            do_adds(b)
            dst_rows = pl.ds((start + r0 + b) * CHUNK, CHUNK)
            wps.append(pltpu.async_copy(buf_a.at[b], out.at[dst_rows], sem_w))
        for wp in wps:
            wp.wait()
        return carry

    lax.fori_loop(0, NGRP, group, 0, unroll=False)

    @pl.when(nloc == IDXR)
    def _tail():
        r = IDXR - 1
        cpa = pltpu.async_copy(ta.at[idx_a.at[r]], buf_a.at[0], sem_g)
        cpb = pltpu.async_copy(tb.at[idx_b.at[r]], buf_b.at[0], sem_g)
        cpa.wait()
        cpb.wait()
        do_adds(0)
        pltpu.sync_copy(buf_a.at[0], out.at[pl.ds((start + r) * CHUNK, CHUNK)])


_sc_gather = pl.kernel(
    _sc_gather_body,
    out_type=jax.ShapeDtypeStruct((N_EDGES, HID), _F32),
    mesh=_MESH,
    compiler_params=pltpu.CompilerParams(use_tc_tiling_on_sc=False),
    scratch_types=[
        pltpu.VMEM((IDXR, CHUNK), jnp.int32),
        pltpu.VMEM((IDXR, CHUNK), jnp.int32),
        pltpu.VMEM((KG, CHUNK, HID), _F32),
        pltpu.VMEM((KG, CHUNK, HID), _F32),
        pltpu.SemaphoreType.DMA,
        pltpu.SemaphoreType.DMA,
    ],
)


def _sc_scatter_body(e_hbm, dst2d, out, acc, idx, buf, zbuf, sem_l, sem_s):
    cid, sid, start, nloc = _worker_range()

    def zrow(i, carry):
        zbuf[i, :] = jnp.zeros((EDGE_DIM,), _F32)
        return carry

    lax.fori_loop(0, ROWS_PER_TILE, zrow, 0, unroll=False)
    rows = pl.ds(sid * ROWS_PER_TILE, ROWS_PER_TILE)
    pltpu.sync_copy(zbuf, acc.at[rows])
    pltpu.sync_copy(dst2d.at[pl.ds(start, IDXR)], idx)
    plsc.subcore_barrier()

    def group(g, carry):
        r0 = g * KG
        cps = []
        for b in range(KG):
            src_rows = pl.ds((start + r0 + b) * CHUNK, CHUNK)
            cps.append(pltpu.async_copy(e_hbm.at[src_rows], buf.at[b], sem_l))
        for cp in cps:
            cp.wait()
        sps = []
        for b in range(KG):
            sps.append(pltpu.async_copy(buf.at[b], acc.at[idx.at[r0 + b]],
                                        sem_s, add=True))
        for sp in sps:
            sp.wait()
        return carry

    lax.fori_loop(0, NGRP, group, 0, unroll=False)

    @pl.when(nloc == IDXR)
    def _tail():
        r = IDXR - 1
        pltpu.sync_copy(e_hbm.at[pl.ds((start + r) * CHUNK, CHUNK)], buf.at[0])
        pltpu.sync_copy(buf.at[0], acc.at[idx.at[r]], add=True)

    plsc.subcore_barrier()
    pltpu.sync_copy(acc.at[rows], out.at[cid, rows])


_sc_scatter = pl.kernel(
    _sc_scatter_body,
    out_type=jax.ShapeDtypeStruct((NC, N_NODES, EDGE_DIM), _F32),
    mesh=_MESH,
    compiler_params=pltpu.CompilerParams(use_tc_tiling_on_sc=False),
    scratch_types=[
        pltpu.VMEM_SHARED((N_NODES, EDGE_DIM), _F32),
        pltpu.VMEM((IDXR, CHUNK), jnp.int32),
        pltpu.VMEM((KG, CHUNK, EDGE_DIM), _F32),
        pltpu.VMEM((ROWS_PER_TILE, EDGE_DIM), _F32),
        pltpu.SemaphoreType.DMA,
        pltpu.SemaphoreType.DMA,
    ],
)


def _layer(x, ea, dst2d, src2d, p):
    rel, obj = p["rel"], p["obj"]
    pad = HID - rel["W1"].shape[1]
    a = jnp.pad(rel["W1"][:NODE_DIM], ((0, 0), (0, pad)))
    b = jnp.pad(rel["W1"][NODE_DIM:2 * NODE_DIM], ((0, 0), (0, pad)))
    cp = jnp.pad(rel["W1"][2 * NODE_DIM:], ((0, 0), (0, pad)))
    wcat = jnp.concatenate([a, b], axis=1)
    b1p = jnp.pad(rel["b1"], (0, pad)).reshape(1, HID)
    w2p = jnp.pad(rel["W2"], ((0, pad), (0, 0)))
    b2p = rel["b2"].reshape(1, EDGE_DIM)

    pa, pb = _node_proj(x, wcat)
    gsum = _sc_gather(pa, pb, dst2d, src2d)
    e_t = _edge_mlp(gsum, ea, cp, b1p, w2p, b2p)
    aggp = _sc_scatter(e_t, dst2d)
    x_new = _node_update(
        x, aggp,
        obj["W1"][:NODE_DIM], obj["W1"][NODE_DIM:],
        obj["b1"].reshape(1, HID_O), obj["W2"], obj["b2"].reshape(1, NODE_DIM))
    return x_new, e_t


def kernel(x, edge_index, edge_attr, params):
    src2d = edge_index[0].reshape(N_CHUNKS, CHUNK)
    dst2d = edge_index[1].reshape(N_CHUNKS, CHUNK)
    ea = edge_attr
    edge_attrs = [edge_attr]
    for p in params:
        x, ea = _layer(x, ea, dst2d, src2d, p)
        edge_attrs.append(ea)
    return x, ea, tuple(edge_attrs)


# speedup vs baseline: 3.8365x; 1.0736x over previous
import functools

import jax
import jax.numpy as jnp
from jax import lax
from jax.experimental import pallas as pl
from jax.experimental.pallas import tpu as pltpu
from jax.experimental.pallas import tpu_sc as plsc

N_NODES = 10000
N_EDGES = 320000
NODE_DIM = 128
EDGE_DIM = 16
HID = 48
HID_O = 40
ALPHA = 0.5

CHUNK = 128
N_CHUNKS = N_EDGES // CHUNK
NC, NS = 2, 16
NW = NC * NS
ROWS_PER_TILE = N_NODES // NS

_F32 = jnp.float32


def _nodeproj_body(x_ref, w_ref, pa_ref, pb_ref):
    p = jnp.dot(x_ref[...], w_ref[...], preferred_element_type=_F32)
    pa_ref[...] = p[:, :HID]
    pb_ref[...] = p[:, HID:]


def _node_proj(x, wcat):
    bn = 2000
    return pl.pallas_call(
        _nodeproj_body,
        grid=(N_NODES // bn,),
        in_specs=[
            pl.BlockSpec((bn, NODE_DIM), lambda i: (i, 0)),
            pl.BlockSpec((NODE_DIM, 2 * HID), lambda i: (0, 0)),
        ],
        out_specs=[
            pl.BlockSpec((bn, HID), lambda i: (i, 0)),
            pl.BlockSpec((bn, HID), lambda i: (i, 0)),
        ],
        out_shape=[jax.ShapeDtypeStruct((N_NODES, HID), _F32)] * 2,
    )(x, wcat)


def _edge_body(gt_ref, eat_ref, cpt_ref, b1_ref, w2t_ref, b2_ref, out_ref):
    m = jnp.dot(cpt_ref[...], eat_ref[...], preferred_element_type=_F32)
    h = jnp.maximum(gt_ref[...] + m + b1_ref[...].T, 0.0)
    out_ref[...] = (jnp.dot(w2t_ref[...], h, preferred_element_type=_F32)
                    + b2_ref[...].T)


def _edge_mlp(gsum_t, ea_t, cpt, b1p, w2t, b2p):
    be = 2560
    return pl.pallas_call(
        _edge_body,
        grid=(N_EDGES // be,),
        in_specs=[
            pl.BlockSpec((HID, be), lambda i: (0, i)),
            pl.BlockSpec((EDGE_DIM, be), lambda i: (0, i)),
            pl.BlockSpec((HID, EDGE_DIM), lambda i: (0, 0)),
            pl.BlockSpec((1, HID), lambda i: (0, 0)),
            pl.BlockSpec((EDGE_DIM, HID), lambda i: (0, 0)),
            pl.BlockSpec((1, EDGE_DIM), lambda i: (0, 0)),
        ],
        out_specs=pl.BlockSpec((EDGE_DIM, be), lambda i: (0, i)),
        out_shape=jax.ShapeDtypeStruct((EDGE_DIM, N_EDGES), _F32),
    )(gsum_t, ea_t, cpt, b1p, w2t, b2p)


def _nodeupd_body(x_ref, aggp_ref, o1x_ref, o1a_ref, ob1_ref, o2_ref, ob2_ref,
                  out_ref):
    agg = aggp_ref[0] + aggp_ref[1]
    t = (jnp.dot(x_ref[...], o1x_ref[...], preferred_element_type=_F32)
         + jnp.dot(agg, o1a_ref[...], preferred_element_type=_F32)
         + ob1_ref[...])
    h = jnp.maximum(t, 0.0)
    delta = jnp.dot(h, o2_ref[...], preferred_element_type=_F32) + ob2_ref[...]
    out_ref[...] = ALPHA * x_ref[...] + (1.0 - ALPHA) * jnp.maximum(delta, 0.0)


def _node_update(x, aggp, o1x, o1a, ob1, o2, ob2):
    bn = 2000
    return pl.pallas_call(
        _nodeupd_body,
        grid=(N_NODES // bn,),
        in_specs=[
            pl.BlockSpec((bn, NODE_DIM), lambda i: (i, 0)),
            pl.BlockSpec((2, bn, EDGE_DIM), lambda i: (0, i, 0)),
            pl.BlockSpec((NODE_DIM, HID_O), lambda i: (0, 0)),
            pl.BlockSpec((EDGE_DIM, HID_O), lambda i: (0, 0)),
            pl.BlockSpec((1, HID_O), lambda i: (0, 0)),
            pl.BlockSpec((HID_O, NODE_DIM), lambda i: (0, 0)),
            pl.BlockSpec((1, NODE_DIM), lambda i: (0, 0)),
        ],
        out_specs=pl.BlockSpec((bn, NODE_DIM), lambda i: (i, 0)),
        out_shape=jax.ShapeDtypeStruct((N_NODES, NODE_DIM), _F32),
    )(x, aggp, o1x, o1a, ob1, o2, ob2)


_MESH = plsc.VectorSubcoreMesh(core_axis_name="c", subcore_axis_name="s")

IDXR = 79
KG = 6
NGRP = 78 // KG
KS = 6
NGRPS = 78 // KS


def _worker_range():
    cid = lax.axis_index("c")
    sid = lax.axis_index("s")
    wid = cid * NS + sid
    start = (wid * N_CHUNKS) // NW
    nloc = ((wid + 1) * N_CHUNKS) // NW - start
    return cid, sid, start, nloc


def _sc_gather_body(ta, tb, dst2d, src2d, out, idx_a, idx_b,
                    buf_a, buf_b, sem_g, sem_w):
    cid, sid, start, nloc = _worker_range()

    pltpu.sync_copy(dst2d.at[pl.ds(start, IDXR)], idx_a)
    pltpu.sync_copy(src2d.at[pl.ds(start, IDXR)], idx_b)

    def do_adds(b):
        def add_row(j, carry2):
            for k in range(HID // 16):
                s = pl.ds(k * 16, 16)
                buf_a[b, j, s] = buf_a[b, j, s] + buf_b[b, j, s]
            return carry2

        lax.fori_loop(0, CHUNK, add_row, 0, unroll=False)

    def group(g, carry):
        r0 = g * KG
        cps = []
        for b in range(KG):
            cps.append(pltpu.async_copy(ta.at[idx_a.at[r0 + b]],
                                        buf_a.at[b], sem_g))
            cps.append(pltpu.async_copy(tb.at[idx_b.at[r0 + b]],
                                        buf_b.at[b], sem_g))
        for cp in cps:
            cp.wait()
        wps = []
        for b in range(KG):
            do_adds(b)
            dst_rows = pl.ds((start + r0 + b) * CHUNK, CHUNK)
            wps.append(pltpu.async_copy(buf_a.at[b], out.at[dst_rows], sem_w))
        for wp in wps:
            wp.wait()
        return carry

    lax.fori_loop(0, NGRP, group, 0, unroll=False)

    @pl.when(nloc == IDXR)
    def _tail():
        r = IDXR - 1
        cpa = pltpu.async_copy(ta.at[idx_a.at[r]], buf_a.at[0], sem_g)
        cpb = pltpu.async_copy(tb.at[idx_b.at[r]], buf_b.at[0], sem_g)
        cpa.wait()
        cpb.wait()
        do_adds(0)
        pltpu.sync_copy(buf_a.at[0], out.at[pl.ds((start + r) * CHUNK, CHUNK)])


_sc_gather = pl.kernel(
    _sc_gather_body,
    out_type=jax.ShapeDtypeStruct((N_EDGES, HID), _F32),
    mesh=_MESH,
    compiler_params=pltpu.CompilerParams(use_tc_tiling_on_sc=False),
    scratch_types=[
        pltpu.VMEM((IDXR, CHUNK), jnp.int32),
        pltpu.VMEM((IDXR, CHUNK), jnp.int32),
        pltpu.VMEM((KG, CHUNK, HID), _F32),
        pltpu.VMEM((KG, CHUNK, HID), _F32),
        pltpu.SemaphoreType.DMA,
        pltpu.SemaphoreType.DMA,
    ],
)


def _sc_scatter_body(e_hbm, dst2d, out, acc, idx, buf, zbuf, sem_l, sem_s):
    cid, sid, start, nloc = _worker_range()

    def zrow(i, carry):
        zbuf[i, :] = jnp.zeros((EDGE_DIM,), _F32)
        return carry

    lax.fori_loop(0, ROWS_PER_TILE, zrow, 0, unroll=False)
    rows = pl.ds(sid * ROWS_PER_TILE, ROWS_PER_TILE)
    pltpu.sync_copy(zbuf, acc.at[rows])
    pltpu.sync_copy(dst2d.at[pl.ds(start, IDXR)], idx)
    plsc.subcore_barrier()

    def group(g, carry):
        r0 = g * KS
        cps = []
        for b in range(KS):
            src_rows = pl.ds((start + r0 + b) * CHUNK, CHUNK)
            cps.append(pltpu.async_copy(e_hbm.at[src_rows], buf.at[b], sem_l))
        for cp in cps:
            cp.wait()
        sps = []
        for b in range(KS):
            sps.append(pltpu.async_copy(buf.at[b], acc.at[idx.at[r0 + b]],
                                        sem_s, add=True))
        for sp in sps:
            sp.wait()
        return carry

    lax.fori_loop(0, NGRPS, group, 0, unroll=False)

    @pl.when(nloc == IDXR)
    def _tail():
        r = IDXR - 1
        pltpu.sync_copy(e_hbm.at[pl.ds((start + r) * CHUNK, CHUNK)], buf.at[0])
        pltpu.sync_copy(buf.at[0], acc.at[idx.at[r]], add=True)

    plsc.subcore_barrier()
    pltpu.sync_copy(acc.at[rows], out.at[cid, rows])


_sc_scatter = pl.kernel(
    _sc_scatter_body,
    out_type=jax.ShapeDtypeStruct((NC, N_NODES, EDGE_DIM), _F32),
    mesh=_MESH,
    compiler_params=pltpu.CompilerParams(use_tc_tiling_on_sc=False),
    scratch_types=[
        pltpu.VMEM_SHARED((N_NODES, EDGE_DIM), _F32),
        pltpu.VMEM((IDXR, CHUNK), jnp.int32),
        pltpu.VMEM((KS, CHUNK, EDGE_DIM), _F32),
        pltpu.VMEM((ROWS_PER_TILE, EDGE_DIM), _F32),
        pltpu.SemaphoreType.DMA,
        pltpu.SemaphoreType.DMA,
    ],
)


def _layer(x, ea, dst2d, src2d, p):
    rel, obj = p["rel"], p["obj"]
    pad = HID - rel["W1"].shape[1]
    a = jnp.pad(rel["W1"][:NODE_DIM], ((0, 0), (0, pad)))
    b = jnp.pad(rel["W1"][NODE_DIM:2 * NODE_DIM], ((0, 0), (0, pad)))
    cpt = jnp.pad(rel["W1"][2 * NODE_DIM:], ((0, 0), (0, pad))).T
    wcat = jnp.concatenate([a, b], axis=1)
    b1p = jnp.pad(rel["b1"], (0, pad)).reshape(1, HID)
    w2t = jnp.pad(rel["W2"], ((0, pad), (0, 0))).T
    b2p = rel["b2"].reshape(1, EDGE_DIM)

    pa, pb = _node_proj(x, wcat)
    gsum = _sc_gather(pa, pb, dst2d, src2d)
    e_t = _edge_mlp(gsum.T, ea, cpt, b1p, w2t, b2p)
    aggp = _sc_scatter(e_t.T, dst2d)
    x_new = _node_update(
        x, aggp,
        obj["W1"][:NODE_DIM], obj["W1"][NODE_DIM:],
        obj["b1"].reshape(1, HID_O), obj["W2"], obj["b2"].reshape(1, NODE_DIM))
    return x_new, e_t


def kernel(x, edge_index, edge_attr, params):
    src2d = edge_index[0].reshape(N_CHUNKS, CHUNK)
    dst2d = edge_index[1].reshape(N_CHUNKS, CHUNK)
    ea_t = edge_attr.T
    e_ts = []
    for p in params:
        x, ea_t = _layer(x, ea_t, dst2d, src2d, p)
        e_ts.append(ea_t)
    e1, e2 = e_ts[0].T, e_ts[1].T
    return x, e2, (edge_attr, e1, e2)
